# baseline (speedup 1.0000x reference)
import jax
import jax.numpy as jnp
from jax.experimental import pallas as pl
from jax.experimental.pallas import tpu as pltpu

BM = 256
NCH = 64


def _matmul_kernel(x_ref, w_hbm, b_ref, o_ref, w16_ref, wtmp, sem):
    K = w_hbm.shape[0]
    CK = K // NCH

    @pl.when(pl.program_id(0) == 0)
    def _load_w():
        def _start(c):
            return pltpu.make_async_copy(
                w_hbm.at[pl.ds(c * CK, CK), :], wtmp.at[c % 2], sem.at[c % 2]
            )

        _start(0).start()
        for c in range(NCH):
            if c + 1 < NCH:
                _start(c + 1).start()
            _start(c).wait()
            w16_ref[pl.ds(c * CK, CK), :] = wtmp[c % 2].astype(jnp.bfloat16)

    o_ref[...] = (
        jax.lax.dot_general(
            x_ref[...],
            w16_ref[...],
            (((1,), (0,)), ((), ())),
            preferred_element_type=jnp.float32,
        )
        + b_ref[...]
    )


def kernel(inputs, W, b):
    M, K = inputs.shape
    _, N = W.shape
    b2d = b.reshape(1, N)

    grid = (M // BM,)
    out = pl.pallas_call(
        _matmul_kernel,
        grid=grid,
        in_specs=[
            pl.BlockSpec((BM, K), lambda i: (i, 0)),
            pl.BlockSpec(memory_space=pl.ANY),
            pl.BlockSpec((1, N), lambda i: (0, 0)),
        ],
        out_specs=pl.BlockSpec((BM, N), lambda i: (i, 0)),
        out_shape=jax.ShapeDtypeStruct((M, N), jnp.float32),
        scratch_shapes=[
            pltpu.VMEM((K, N), jnp.bfloat16),
            pltpu.VMEM((2, K // NCH, N), jnp.float32),
            pltpu.SemaphoreType.DMA((2,)),
        ],
        compiler_params=pltpu.CompilerParams(
            dimension_semantics=("arbitrary",),
        ),
    )(inputs, W, b2d)
    return out


# speedup vs baseline: 1.0746x; 1.0746x over previous
import jax
import jax.numpy as jnp
from jax.experimental import pallas as pl
from jax.experimental.pallas import tpu as pltpu

BM = 256
NCH = 16


def _matmul_kernel(x_ref, w_hbm, b_ref, o_ref, w16_ref, wtmp, sem):
    K = w_hbm.shape[0]
    CK = K // NCH

    @pl.when(pl.program_id(0) == 0)
    def _load_w():
        def _start(c):
            return pltpu.make_async_copy(
                w_hbm.at[pl.ds(c * CK, CK), :], wtmp.at[c % 2], sem.at[c % 2]
            )

        _start(0).start()
        for c in range(NCH):
            if c + 1 < NCH:
                _start(c + 1).start()
            _start(c).wait()
            w16_ref[pl.ds(c * CK, CK), :] = wtmp[c % 2].astype(jnp.bfloat16)

    o_ref[...] = (
        jax.lax.dot_general(
            x_ref[...],
            w16_ref[...],
            (((1,), (0,)), ((), ())),
            preferred_element_type=jnp.float32,
        )
        + b_ref[...]
    )


def kernel(inputs, W, b):
    M, K = inputs.shape
    _, N = W.shape
    b2d = b.reshape(1, N)

    grid = (M // BM,)
    out = pl.pallas_call(
        _matmul_kernel,
        grid=grid,
        in_specs=[
            pl.BlockSpec((BM, K), lambda i: (i, 0)),
            pl.BlockSpec(memory_space=pl.ANY),
            pl.BlockSpec((1, N), lambda i: (0, 0)),
        ],
        out_specs=pl.BlockSpec((BM, N), lambda i: (i, 0)),
        out_shape=jax.ShapeDtypeStruct((M, N), jnp.float32),
        scratch_shapes=[
            pltpu.VMEM((K, N), jnp.bfloat16),
            pltpu.VMEM((2, K // NCH, N), jnp.float32),
            pltpu.SemaphoreType.DMA((2,)),
        ],
        compiler_params=pltpu.CompilerParams(
            dimension_semantics=("arbitrary",),
        ),
    )(inputs, W, b2d)
    return out


# speedup vs baseline: 1.0748x; 1.0002x over previous
import jax
import jax.numpy as jnp
from jax.experimental import pallas as pl
from jax.experimental.pallas import tpu as pltpu

BM = 256
NCH = 16


def _matmul_kernel(x_ref, w_hbm, b_ref, o_ref, w16_ref, wtmp, sem):
    K = w_hbm.shape[0]
    CK = K // NCH

    @pl.when(pl.program_id(0) == 0)
    def _load_w():
        def _start(c):
            return pltpu.make_async_copy(
                w_hbm.at[pl.ds(c * CK, CK), :], wtmp.at[c % 2], sem.at[c % 2]
            )

        _start(0).start()
        for c in range(NCH):
            if c + 1 < NCH:
                _start(c + 1).start()
            _start(c).wait()
            w16_ref[pl.ds(c * CK, CK), :] = wtmp[c % 2].astype(jnp.bfloat16)

    o_ref[...] = (
        jax.lax.dot_general(
            x_ref[...],
            w16_ref[...],
            (((1,), (0,)), ((), ())),
            preferred_element_type=jnp.float32,
        )
        + b_ref[...]
    )


def kernel(inputs, W, b):
    M, K = inputs.shape
    _, N = W.shape
    b2d = b.reshape(1, N)

    grid = (M // BM,)
    out = pl.pallas_call(
        _matmul_kernel,
        grid=grid,
        in_specs=[
            pl.BlockSpec((BM, K), lambda i: (i, 0)),
            pl.BlockSpec(memory_space=pl.ANY),
            pl.BlockSpec((1, N), lambda i: (0, 0)),
        ],
        out_specs=pl.BlockSpec((BM, N), lambda i: (i, 0)),
        out_shape=jax.ShapeDtypeStruct((M, N), jnp.float32),
        scratch_shapes=[
            pltpu.VMEM((K, N), jnp.bfloat16),
            pltpu.VMEM((2, K // NCH, N), jnp.float32),
            pltpu.SemaphoreType.DMA((2,)),
        ],
        compiler_params=pltpu.CompilerParams(
            dimension_semantics=("parallel",),
        ),
    )(inputs, W, b2d)
    return out
